# Initial kernel scaffold; baseline (speedup 1.0000x reference)
import jax
import jax.numpy as jnp
from jax.experimental import pallas as pl


def kernel(text_features, W1, b1, W2, b2, W3, b3):
    raise NotImplementedError("write your pallas kernel here")


# speedup vs baseline: 1.0768x; 1.0768x over previous
import functools

import jax
import jax.numpy as jnp
from jax.experimental import pallas as pl
from jax.experimental.pallas import tpu as pltpu

B, L, DIM = 4, 2048, 4096
NUM_LAYERS, TOP = 24, 5
CK = 256
KSTEPS = DIM // CK


def _router_body(x_ref, w1_ref, w2_ref, w3_ref, b1_ref, b2_ref, b3_ref,
                 idx_ref, wts_ref, probs_ref, acc1_ref, acc2_ref):
    i = pl.program_id(0)

    pooled_c = jnp.sum(x_ref[...], axis=1) * (1.0 / L)

    p1 = jax.lax.dot_general(pooled_c, w1_ref[...],
                             (((1,), (1,)), ((), ())),
                             preferred_element_type=jnp.float32)
    p2 = jax.lax.dot_general(pooled_c, w2_ref[...],
                             (((1,), (1,)), ((), ())),
                             preferred_element_type=jnp.float32)

    @pl.when(i == 0)
    def _init():
        acc1_ref[...] = p1
        acc2_ref[...] = p2

    @pl.when(i > 0)
    def _acc():
        acc1_ref[...] += p1
        acc2_ref[...] += p2

    @pl.when(i == KSTEPS - 1)
    def _epilogue():
        h1 = jax.nn.silu(acc1_ref[...] + b1_ref[...])
        h2 = jax.nn.silu(acc2_ref[...] + b2_ref[...])
        gated = h1 * h2
        logits = jax.lax.dot_general(gated, w3_ref[...],
                                     (((1,), (1,)), ((), ())),
                                     preferred_element_type=jnp.float32)
        logits = logits + b3_ref[...]

        m = jnp.max(logits, axis=-1, keepdims=True)
        e = jnp.exp(logits - m)
        probs = e / jnp.sum(e, axis=-1, keepdims=True)
        probs_ref[...] = probs

        iota = jax.lax.broadcasted_iota(jnp.int32, (B, NUM_LAYERS), 1)
        work = probs
        idx_cols = []
        wt_cols = []
        for _ in range(TOP):
            mv = jnp.max(work, axis=-1, keepdims=True)
            is_max = work == mv
            sel = jnp.min(jnp.where(is_max, iota, NUM_LAYERS * 2),
                          axis=-1, keepdims=True)
            idx_cols.append(sel)
            wt_cols.append(mv)
            work = jnp.where(iota == sel, -jnp.inf, work)
        top_w = jnp.concatenate(wt_cols, axis=1)
        top_i = jnp.concatenate(idx_cols, axis=1)
        idx_ref[...] = top_i
        wts_ref[...] = top_w / jnp.sum(top_w, axis=-1, keepdims=True)


@jax.jit
def _router(text_features, W1, W2, W3, b1, b2, b3):
    grid = (KSTEPS,)
    kernel_fn = pl.pallas_call(
        _router_body,
        grid=grid,
        in_specs=[
            pl.BlockSpec((B, L, CK), lambda i: (0, 0, i)),
            pl.BlockSpec((DIM, CK), lambda i: (0, i)),
            pl.BlockSpec((DIM, CK), lambda i: (0, i)),
            pl.BlockSpec((NUM_LAYERS, DIM), lambda i: (0, 0)),
            pl.BlockSpec((1, DIM), lambda i: (0, 0)),
            pl.BlockSpec((1, DIM), lambda i: (0, 0)),
            pl.BlockSpec((1, NUM_LAYERS), lambda i: (0, 0)),
        ],
        out_specs=[
            pl.BlockSpec((B, TOP), lambda i: (0, 0)),
            pl.BlockSpec((B, TOP), lambda i: (0, 0)),
            pl.BlockSpec((B, NUM_LAYERS), lambda i: (0, 0)),
        ],
        out_shape=[
            jax.ShapeDtypeStruct((B, TOP), jnp.int32),
            jax.ShapeDtypeStruct((B, TOP), jnp.float32),
            jax.ShapeDtypeStruct((B, NUM_LAYERS), jnp.float32),
        ],
        scratch_shapes=[
            pltpu.VMEM((B, DIM), jnp.float32),
            pltpu.VMEM((B, DIM), jnp.float32),
        ],
        compiler_params=pltpu.CompilerParams(
            dimension_semantics=("arbitrary",),
        ),
    )
    return kernel_fn(text_features, W1, W2, W3,
                     b1[None, :], b2[None, :], b3[None, :])


def kernel(text_features, W1, b1, W2, b2, W3, b3):
    top_i, top_w, probs = _router(text_features, W1, W2, W3, b1, b2, b3)
    return (top_i, top_w, probs)
